# Initial kernel scaffold; baseline (speedup 1.0000x reference)
import jax
import jax.numpy as jnp
from jax.experimental import pallas as pl


def kernel(x_num, x_cat, cat_table, num_weight, num_bias):
    raise NotImplementedError("write your pallas kernel here")


# speedup vs baseline: 1.7296x; 1.7296x over previous
import functools

import jax
import jax.numpy as jnp
from jax import lax
from jax.experimental import pallas as pl
from jax.experimental.pallas import tpu as pltpu
from jax.experimental.pallas import tpu_sc as plsc

_NUM_CORES = 2
_NUM_SUBCORES = 16
_NUM_WORKERS = _NUM_CORES * _NUM_SUBCORES

_CHUNK = 1024


def _sc_gather(table, idx):
    dim = table.shape[1]
    n = idx.shape[0]
    per_w = n // _NUM_WORKERS
    n_chunks = per_w // _CHUNK
    assert per_w % _CHUNK == 0, (n, per_w)
    idx3 = idx.reshape(n // _CHUNK, _CHUNK)

    mesh = plsc.VectorSubcoreMesh(core_axis_name="c", subcore_axis_name="s")

    @functools.partial(
        pl.kernel,
        out_type=jax.ShapeDtypeStruct((n // _CHUNK, _CHUNK, dim), jnp.float32),
        mesh=mesh,
        scratch_types=[
            pltpu.VMEM((2, _CHUNK), jnp.int32),
            pltpu.VMEM((2, _CHUNK, dim), jnp.float32),
            pltpu.SemaphoreType.DMA,
        ],
        compiler_params=pltpu.CompilerParams(use_tc_tiling_on_sc=False),
    )
    def k(table_hbm, idx_hbm, out_hbm, idx_v, rows_v, gsem):
        wid = lax.axis_index("s") * _NUM_CORES + lax.axis_index("c")
        chunk_base = wid * n_chunks

        def load_idx(g, buf):
            pltpu.sync_copy(idx_hbm.at[chunk_base + g], idx_v.at[buf])

        def start_gather(buf):
            return pltpu.async_copy(
                table_hbm.at[idx_v.at[buf]], rows_v.at[buf], gsem
            )

        def store_out(g, buf):
            pltpu.sync_copy(
                rows_v.at[buf],
                out_hbm.at[chunk_base + g],
            )

        load_idx(0, 0)
        pending = start_gather(0)
        for g in range(1, n_chunks):
            buf = g % 2
            load_idx(g, buf)
            nxt = start_gather(buf)
            pending.wait()
            store_out(g - 1, 1 - buf)
            pending = nxt
        pending.wait()
        store_out(n_chunks - 1, (n_chunks - 1) % 2)

    return k(table, idx3)


def _tc_assemble(x_num, wmat, bflat, cat2d):
    b, n_num = x_num.shape
    ncd = cat2d.shape[1]
    nnd = wmat.shape[1]
    blk = 1024
    grid = (b // blk,)

    def body(x_ref, w_ref, b_ref, cat_ref, out_ref):
        num = (
            jnp.dot(x_ref[...], w_ref[...], preferred_element_type=jnp.float32)
            + b_ref[...]
        )
        out_ref[:, :nnd] = num
        out_ref[:, nnd:] = cat_ref[...]

    return pl.pallas_call(
        body,
        grid=grid,
        in_specs=[
            pl.BlockSpec((blk, n_num), lambda i: (i, 0)),
            pl.BlockSpec((n_num, nnd), lambda i: (0, 0)),
            pl.BlockSpec((1, nnd), lambda i: (0, 0)),
            pl.BlockSpec((blk, ncd), lambda i: (i, 0)),
        ],
        out_specs=pl.BlockSpec((blk, nnd + ncd), lambda i: (i, 0)),
        out_shape=jax.ShapeDtypeStruct((b, nnd + ncd), jnp.float32),
    )(x_num, wmat, bflat, cat2d)


def kernel(x_num, x_cat, cat_table, num_weight, num_bias):
    b, n_cat = x_cat.shape
    n_num = x_num.shape[1]
    dim = cat_table.shape[1]
    idx = x_cat.astype(jnp.int32).reshape(-1)
    cat2d = _sc_gather(cat_table, idx).reshape(b, n_cat * dim)
    feat = jnp.arange(n_num * dim, dtype=jnp.int32) // dim
    mask = feat[None, :] == jnp.arange(n_num, dtype=jnp.int32)[:, None]
    wmat = jnp.where(mask, num_weight.reshape(-1)[None, :], 0.0)
    bflat = num_bias.reshape(1, n_num * dim)
    out = _tc_assemble(x_num, wmat, bflat, cat2d)
    return out.reshape(b, n_num + n_cat, dim)
